# Initial kernel scaffold; baseline (speedup 1.0000x reference)
import jax
import jax.numpy as jnp
from jax.experimental import pallas as pl


def kernel(token_ids, w):
    raise NotImplementedError("write your pallas kernel here")


# speedup vs baseline: 7.8622x; 7.8622x over previous
import functools

import jax
import jax.numpy as jnp
from jax import lax
from jax.experimental import pallas as pl
from jax.experimental.pallas import tpu as pltpu
from jax.experimental.pallas import tpu_sc as plsc

NUM_CORES = 2
NUM_SUBCORES = 16
NW = NUM_CORES * NUM_SUBCORES
CHUNK = 128
NBUF = 2


@functools.partial(jax.jit, static_argnums=(2, 3))
def _gather_rows(w, idx_flat, n_rows, d):
    b_per_w = n_rows // NW
    n_chunks = b_per_w // CHUNK
    mesh = plsc.VectorSubcoreMesh(core_axis_name="c", subcore_axis_name="s")

    @functools.partial(
        pl.kernel,
        mesh=mesh,
        out_type=jax.ShapeDtypeStruct((n_rows, d), jnp.float32),
        scratch_types=[
            pltpu.VMEM((b_per_w,), jnp.int32),
            pltpu.VMEM((CHUNK, d), jnp.float32),
            pltpu.VMEM((CHUNK, d), jnp.float32),
            pltpu.SemaphoreType.DMA,
            pltpu.SemaphoreType.DMA,
        ],
    )
    def k(table_hbm, idx_hbm, out_hbm, idx_v, buf0, buf1, sem0, sem1):
        wid = lax.axis_index("s") * NUM_CORES + lax.axis_index("c")
        base = pl.multiple_of(wid * b_per_w, 8)
        pltpu.sync_copy(idx_hbm.at[pl.ds(base, b_per_w)], idx_v)
        bufs = (buf0, buf1)
        sems = (sem0, sem1)
        for b in range(NBUF):
            pltpu.async_copy(
                table_hbm.at[idx_v.at[pl.ds(b * CHUNK, CHUNK)]], bufs[b], sems[b]
            )

        def body(i, carry):
            for b in range(NBUF):
                c = i * NBUF + b
                pltpu.make_async_copy(
                    table_hbm.at[pl.ds(0, CHUNK)], bufs[b], sems[b]
                ).wait()
                off = pl.multiple_of(base + c * CHUNK, 8)
                pltpu.sync_copy(bufs[b], out_hbm.at[pl.ds(off, CHUNK)])
                pltpu.async_copy(
                    table_hbm.at[idx_v.at[pl.ds((c + NBUF) * CHUNK, CHUNK)]],
                    bufs[b],
                    sems[b],
                )
            return carry

        lax.fori_loop(0, (n_chunks - NBUF) // NBUF, body, 0)
        for b in range(NBUF):
            c = n_chunks - NBUF + b
            pltpu.make_async_copy(
                table_hbm.at[pl.ds(0, CHUNK)], bufs[b], sems[b]
            ).wait()
            off = pl.multiple_of(base + c * CHUNK, 8)
            pltpu.sync_copy(bufs[b], out_hbm.at[pl.ds(off, CHUNK)])

    return k(w, idx_flat)


def kernel(token_ids, w):
    n_rows = token_ids.size
    d = w.shape[1]
    idx_flat = token_ids.reshape(-1).astype(jnp.int32)
    out = _gather_rows(w, idx_flat, n_rows, d)
    return out.reshape(*token_ids.shape, d)


# speedup vs baseline: 8.0062x; 1.0183x over previous
import functools

import jax
import jax.numpy as jnp
from jax import lax
from jax.experimental import pallas as pl
from jax.experimental.pallas import tpu as pltpu
from jax.experimental.pallas import tpu_sc as plsc

NUM_CORES = 2
NUM_SUBCORES = 16
NW = NUM_CORES * NUM_SUBCORES
CHUNK = 128
NBUF = 5


@functools.partial(jax.jit, static_argnums=(2, 3))
def _gather_rows(w, idx_flat, n_rows, d):
    b_per_w = n_rows // NW
    n_chunks = b_per_w // CHUNK
    mesh = plsc.VectorSubcoreMesh(core_axis_name="c", subcore_axis_name="s")

    @functools.partial(
        pl.kernel,
        mesh=mesh,
        out_type=jax.ShapeDtypeStruct((n_rows, d), jnp.float32),
        scratch_types=[pltpu.VMEM((b_per_w,), jnp.int32)]
        + [pltpu.VMEM((CHUNK, d), jnp.float32) for _ in range(NBUF)]
        + [pltpu.SemaphoreType.DMA for _ in range(NBUF)],
    )
    def k(table_hbm, idx_hbm, out_hbm, idx_v, *scratch):
        bufs = scratch[:NBUF]
        sems = scratch[NBUF:]
        wid = lax.axis_index("s") * NUM_CORES + lax.axis_index("c")
        base = pl.multiple_of(wid * b_per_w, 8)
        pltpu.sync_copy(idx_hbm.at[pl.ds(base, b_per_w)], idx_v)
        for b in range(NBUF):
            pltpu.async_copy(
                table_hbm.at[idx_v.at[pl.ds(b * CHUNK, CHUNK)]], bufs[b], sems[b]
            )

        def body(i, carry):
            for b in range(NBUF):
                c = i * NBUF + b
                pltpu.make_async_copy(
                    table_hbm.at[pl.ds(0, CHUNK)], bufs[b], sems[b]
                ).wait()
                off = pl.multiple_of(base + c * CHUNK, 8)
                pltpu.sync_copy(bufs[b], out_hbm.at[pl.ds(off, CHUNK)])
                pltpu.async_copy(
                    table_hbm.at[idx_v.at[pl.ds((c + NBUF) * CHUNK, CHUNK)]],
                    bufs[b],
                    sems[b],
                )
            return carry

        lax.fori_loop(0, (n_chunks - NBUF) // NBUF, body, 0)
        for b in range(NBUF):
            c = n_chunks - NBUF + b
            pltpu.make_async_copy(
                table_hbm.at[pl.ds(0, CHUNK)], bufs[b], sems[b]
            ).wait()
            off = pl.multiple_of(base + c * CHUNK, 8)
            pltpu.sync_copy(bufs[b], out_hbm.at[pl.ds(off, CHUNK)])

    return k(w, idx_flat)


def kernel(token_ids, w):
    n_rows = token_ids.size
    d = w.shape[1]
    idx_flat = token_ids.reshape(-1).astype(jnp.int32)
    out = _gather_rows(w, idx_flat, n_rows, d)
    return out.reshape(*token_ids.shape, d)
